# baseline (speedup 1.0000x reference)
import jax
import jax.numpy as jnp
from jax.experimental import pallas as pl


_BLOCK_S = 256


def _add_kernel(x_ref, w_ref, o_ref):
    o_ref[...] = x_ref[...] + w_ref[...][None, :, :]


def kernel(inputs, weight):
    batch, seq_len, dim = inputs.shape
    bs = min(_BLOCK_S, seq_len)
    grid = (seq_len // bs,)
    return pl.pallas_call(
        _add_kernel,
        grid=grid,
        in_specs=[
            pl.BlockSpec((batch, bs, dim), lambda i: (0, i, 0)),
            pl.BlockSpec((bs, dim), lambda i: (i, 0)),
        ],
        out_specs=pl.BlockSpec((batch, bs, dim), lambda i: (0, i, 0)),
        out_shape=jax.ShapeDtypeStruct((batch, seq_len, dim), inputs.dtype),
    )(inputs, weight[:seq_len])


# speedup vs baseline: 1.0004x; 1.0004x over previous
import jax
import jax.numpy as jnp
from jax.experimental import pallas as pl


_BLOCK_S = 128


def _add_kernel(x_ref, w_ref, o_ref):
    o_ref[...] = x_ref[...] + w_ref[...][None, :, :]


def kernel(inputs, weight):
    batch, seq_len, dim = inputs.shape
    bs = min(_BLOCK_S, seq_len)
    grid = (seq_len // bs,)
    return pl.pallas_call(
        _add_kernel,
        grid=grid,
        in_specs=[
            pl.BlockSpec((batch, bs, dim), lambda i: (0, i, 0)),
            pl.BlockSpec((bs, dim), lambda i: (i, 0)),
        ],
        out_specs=pl.BlockSpec((batch, bs, dim), lambda i: (0, i, 0)),
        out_shape=jax.ShapeDtypeStruct((batch, seq_len, dim), inputs.dtype),
    )(inputs, weight[:seq_len])
